# baseline (speedup 1.0000x reference)
import dataclasses

import jax
import jax.numpy as jnp
from jax import lax
from jax.experimental import pallas as pl
from jax.experimental.pallas import tpu as pltpu
from jax.experimental.pallas import tpu_sc as plsc

T = 26
V = 100000
D = 16
B = 16384
NC, NS = 2, 16
NW = NC * NS
BW = B // NW
IW = 128
KC = BW // IW
NU = T * KC
RING = 4
GPT = V // 8
NT = (T * B) // IW


def _gather_body(keys_hbm, tbl_hbm, out_hbm, kbuf, ibuf, rbuf, tbuf, ksem,
                 gsem, osem):
    cid = lax.axis_index("core")
    sid = lax.axis_index("subcore")
    wid = sid * NC + cid

    pltpu.sync_copy(keys_hbm.at[:, pl.ds(wid * KC, KC)], kbuf)

    lanes = lax.iota(jnp.int32, 16)
    rows = [lanes + (c * 16) for c in range(8)]

    def gather_desc(u, slot):
        return pltpu.make_async_copy(
            tbl_hbm.at[ibuf.at[slot]], rbuf.at[slot], gsem.at[slot])

    def out_desc(u, slot, half):
        t = u // KC
        tile = t * (B // IW) + wid * KC + (u % KC)
        return pltpu.make_async_copy(
            tbuf.at[slot, pl.ds(half * 8, 8)],
            out_hbm.at[pl.ds((half * NT + tile) * 8, 8)], osem.at[slot])

    def fill_ibuf(u, slot):
        t = u // KC
        j = u % KC
        for c in range(8):
            kv = kbuf[t, j, pl.ds(c * 16, 16)]
            ibuf[slot, pl.ds(c * 16, 16)] = (kv >> 3) + t * GPT

    for b in range(RING):
        fill_ibuf(b, b)
        gather_desc(b, b).start()

    @pl.loop(0, NU, step=RING)
    def _group(g):
        for b in range(RING):
            u = g + b
            gather_desc(u, b).wait()

            @pl.when(u >= RING)
            def _():
                out_desc(u - RING, b, 0).wait()
                out_desc(u - RING, b, 1).wait()

            t = u // KC
            j = u % KC
            for c in range(8):
                kv = kbuf[t, j, pl.ds(c * 16, 16)]
                colbase = (kv & 7) * 16
                for d in range(D):
                    tbuf[b, d, pl.ds(c * 16, 16)] = plsc.load_gather(
                        rbuf.at[b], [rows[c], colbase + d])

            out_desc(u, b, 0).start()
            out_desc(u, b, 1).start()

            @pl.when(u + RING < NU)
            def _():
                sb = b
                fill_ibuf(u + RING, sb)
                gather_desc(u + RING, sb).start()

    for b in range(RING):
        out_desc(b, b, 0).wait()
        out_desc(b, b, 1).wait()


def _compiler_params():
    cp = pltpu.CompilerParams(use_tc_tiling_on_sc=True)
    if "needs_layout_passes" in pltpu.CompilerParams.__dataclass_fields__:
        cp = dataclasses.replace(cp, needs_layout_passes=False)
    return cp


def kernel(keys_list, tables):
    keys_r = keys_list.reshape(T, B // IW, IW).astype(jnp.int32)
    tbl_g = tables.reshape((T * V * D) // IW, IW)
    mesh = plsc.VectorSubcoreMesh(core_axis_name="core",
                                  subcore_axis_name="subcore")
    out = pl.kernel(
        _gather_body,
        out_type=jax.ShapeDtypeStruct((2 * NT * 8, IW), jnp.float32),
        mesh=mesh,
        compiler_params=_compiler_params(),
        scratch_types=[
            pltpu.VMEM((T, KC, IW), jnp.int32),
            pltpu.VMEM((RING, IW), jnp.int32),
            pltpu.VMEM((RING, IW, IW), jnp.float32),
            pltpu.VMEM((RING, D, IW), jnp.float32),
            pltpu.SemaphoreType.DMA,
            pltpu.SemaphoreType.DMA((RING,)),
            pltpu.SemaphoreType.DMA((RING,)),
        ],
    )(keys_r, tbl_g)
    return (out.reshape(2, NT, 8, IW).transpose(1, 3, 0, 2)
            .reshape(T * B, D))


# speedup vs baseline: 1.0843x; 1.0843x over previous
import dataclasses

import jax
import jax.numpy as jnp
from jax import lax
from jax.experimental import pallas as pl
from jax.experimental.pallas import tpu as pltpu
from jax.experimental.pallas import tpu_sc as plsc

T = 26
V = 100000
D = 16
B = 16384
NC, NS = 2, 16
NW = NC * NS
BW = B // NW
IW = 128
KC = BW // IW
NU = T * KC
RING = 4
NT = (T * B) // IW


def _gather_body(keys_hbm, tbl_hbm, out_hbm, kbuf, rbuf, tbuf, ksem, gsem,
                 osem):
    cid = lax.axis_index("core")
    sid = lax.axis_index("subcore")
    wid = sid * NC + cid

    pltpu.sync_copy(keys_hbm.at[:, wid], kbuf)

    lanes = lax.iota(jnp.int32, 16)

    def gather_desc(u, slot):
        t = u // KC
        return pltpu.make_async_copy(
            tbl_hbm.at[t].at[kbuf.at[t, u % KC]], rbuf.at[slot], gsem.at[slot])

    def out_desc(u, slot, half):
        t = u // KC
        tile = t * (B // IW) + wid * KC + (u % KC)
        return pltpu.make_async_copy(
            tbuf.at[slot, pl.ds(half * 8, 8), pl.ds(0, IW)],
            out_hbm.at[half, tile], osem.at[slot])

    for b in range(RING):
        gather_desc(b, b).start()

    @pl.loop(0, NU, step=RING)
    def _group(g):
        for b in range(RING):
            u = g + b
            gather_desc(u, b).wait()

            @pl.when(u >= RING)
            def _():
                out_desc(u - RING, b, 0).wait()
                out_desc(u - RING, b, 1).wait()

            for l in range(IW):
                plsc.store_scatter(tbuf.at[b],
                                   [lanes, jnp.full((16,), l, jnp.int32)],
                                   rbuf[b, l])

            out_desc(u, b, 0).start()
            out_desc(u, b, 1).start()

            @pl.when(u + RING < NU)
            def _():
                gather_desc(u + RING, b).start()

    for b in range(RING):
        out_desc(b, b, 0).wait()
        out_desc(b, b, 1).wait()


def _compiler_params():
    cp = pltpu.CompilerParams(use_tc_tiling_on_sc=False)
    if "needs_layout_passes" in pltpu.CompilerParams.__dataclass_fields__:
        cp = dataclasses.replace(cp, needs_layout_passes=False)
    return cp


def kernel(keys_list, tables):
    keys_r = keys_list.reshape(T, NW, KC, IW).astype(jnp.int32)
    mesh = plsc.VectorSubcoreMesh(core_axis_name="core",
                                  subcore_axis_name="subcore")
    out = pl.kernel(
        _gather_body,
        out_type=jax.ShapeDtypeStruct((2, NT, 8, IW), jnp.float32),
        mesh=mesh,
        compiler_params=_compiler_params(),
        scratch_types=[
            pltpu.VMEM((T, KC, IW), jnp.int32),
            pltpu.VMEM((RING, IW, D), jnp.float32),
            pltpu.VMEM((RING, D, 129), jnp.float32),
            pltpu.SemaphoreType.DMA,
            pltpu.SemaphoreType.DMA((RING,)),
            pltpu.SemaphoreType.DMA((RING,)),
        ],
    )(keys_r, tables)
    return out.transpose(1, 3, 0, 2).reshape(T * B, D)


# speedup vs baseline: 1.6061x; 1.4812x over previous
import dataclasses

import jax
import jax.numpy as jnp
from jax import lax
from jax.experimental import pallas as pl
from jax.experimental.pallas import tpu as pltpu
from jax.experimental.pallas import tpu_sc as plsc
from jax.experimental.layout import Format, Layout, with_layout_constraint

T = 26
V = 100000
D = 16
B = 16384
NC, NS = 2, 16
NW = NC * NS
BW = B // NW
IW = 128
KC = BW // IW
NU = T * KC
RING = 4
NT = (T * B) // IW


def _gather_body(keys_hbm, tbl_hbm, out_hbm, kbuf, rbuf, tbuf, ksem, gsem,
                 osem):
    cid = lax.axis_index("core")
    sid = lax.axis_index("subcore")
    wid = sid * NC + cid

    pltpu.sync_copy(keys_hbm.at[:, wid], kbuf)

    lanes = lax.iota(jnp.int32, 16)

    def gather_desc(u, slot):
        t = u // KC
        return pltpu.make_async_copy(
            tbl_hbm.at[t].at[kbuf.at[t, u % KC]], rbuf.at[slot], gsem.at[slot])

    def out_desc(u, slot, half):
        t = u // KC
        tile = t * (B // IW) + wid * KC + (u % KC)
        return pltpu.make_async_copy(
            tbuf.at[slot, pl.ds(half * 8, 8), pl.ds(0, IW)],
            out_hbm.at[half, tile], osem.at[slot])

    for b in range(RING):
        gather_desc(b, b).start()

    @pl.loop(0, NU, step=RING)
    def _group(g):
        for b in range(RING):
            u = g + b
            gather_desc(u, b).wait()

            @pl.when(u >= RING)
            def _():
                out_desc(u - RING, b, 0).wait()
                out_desc(u - RING, b, 1).wait()

            for l in range(IW):
                plsc.store_scatter(tbuf.at[b],
                                   [lanes, jnp.full((16,), l, jnp.int32)],
                                   rbuf[b, l])

            out_desc(u, b, 0).start()
            out_desc(u, b, 1).start()

            @pl.when(u + RING < NU)
            def _():
                gather_desc(u + RING, b).start()

    for b in range(RING):
        out_desc(b, b, 0).wait()
        out_desc(b, b, 1).wait()


def _compiler_params():
    cp = pltpu.CompilerParams(use_tc_tiling_on_sc=False)
    if "needs_layout_passes" in pltpu.CompilerParams.__dataclass_fields__:
        cp = dataclasses.replace(cp, needs_layout_passes=False)
    return cp


def kernel(keys_list, tables):
    keys_r = keys_list.reshape(T, NW, KC, IW).astype(jnp.int32)
    tables = with_layout_constraint(tables, Layout((0, 1, 2), ((8,),)))
    mesh = plsc.VectorSubcoreMesh(core_axis_name="core",
                                  subcore_axis_name="subcore")
    out = pl.kernel(
        _gather_body,
        out_type=jax.ShapeDtypeStruct((2, NT, 8, IW), jnp.float32),
        mesh=mesh,
        compiler_params=_compiler_params(),
        scratch_types=[
            pltpu.VMEM((T, KC, IW), jnp.int32),
            pltpu.VMEM((RING, IW, D), jnp.float32),
            pltpu.VMEM((RING, D, 129), jnp.float32),
            pltpu.SemaphoreType.DMA,
            pltpu.SemaphoreType.DMA((RING,)),
            pltpu.SemaphoreType.DMA((RING,)),
        ],
    )(keys_r, tables)
    return out.transpose(1, 3, 0, 2).reshape(T * B, D)
